# baseline (speedup 1.0000x reference)
import jax
import jax.numpy as jnp
from jax import lax
from jax.experimental import pallas as pl

BATCH = 64
NUM_TOK_PER_IMG = 1024
TOKEN_DIM = 32
NUM_TOKENS = 512
NUM_RQ_STEPS = 2

TOK_TOTAL = BATCH * NUM_TOK_PER_IMG
BLOCK = 4096
NUM_BLOCKS = TOK_TOTAL // BLOCK

_DIMS_NT = (((1,), (1,)), ((), ()))
_DIMS_NN = (((1,), (0,)), ((), ()))


def _vq_step(residual, cb):
    r_sq = jnp.sum(residual * residual, axis=-1, keepdims=True)
    c_sq = jnp.sum(cb * cb, axis=-1)[None, :]
    cross = lax.dot_general(residual.astype(jnp.bfloat16),
                            cb.astype(jnp.bfloat16), _DIMS_NT,
                            preferred_element_type=jnp.float32)
    dists = r_sq - 2.0 * cross + c_sq
    m = jnp.min(dists, axis=-1, keepdims=True)
    onehot = (dists == m).astype(jnp.bfloat16)
    hi = cb.astype(jnp.bfloat16)
    rem = cb - hi.astype(jnp.float32)
    mid = rem.astype(jnp.bfloat16)
    lo = (rem - mid.astype(jnp.float32)).astype(jnp.bfloat16)
    k = lax.broadcasted_iota(jnp.int32, (NUM_TOKENS, 1), 0)
    k_even = (k - (k % 2)).astype(jnp.bfloat16)
    k_par = (k % 2).astype(jnp.bfloat16)
    parts = jnp.concatenate([hi, mid, lo, k_even, k_par], axis=1)
    q3 = lax.dot_general(onehot, parts, _DIMS_NN,
                         preferred_element_type=jnp.float32)
    q = ((q3[:, :TOKEN_DIM] + q3[:, TOKEN_DIM:2 * TOKEN_DIM])
         + q3[:, 2 * TOKEN_DIM:3 * TOKEN_DIM])
    idx = (q3[:, 3 * TOKEN_DIM] + q3[:, 3 * TOKEN_DIM + 1]).astype(jnp.int32)
    return q, idx


def _rvq_block(z_ref, cb_ref, idx0_ref, idx1_ref, vq_ref):
    z = z_ref[...]
    q0, i0 = _vq_step(z, cb_ref[0])
    q1, i1 = _vq_step(z - q0, cb_ref[1])
    z_q = q0 + q1
    idx0_ref[0, 0, :] = i0
    idx1_ref[0, 0, :] = i1
    vq_ref[...] = z + (z_q - z)


@jax.jit
def kernel(z_e, codebooks):
    z_flat = z_e.reshape(TOK_TOTAL, TOKEN_DIM)
    idx0, idx1, vq = pl.pallas_call(
        _rvq_block,
        grid=(NUM_BLOCKS,),
        in_specs=[
            pl.BlockSpec((BLOCK, TOKEN_DIM), lambda i: (i, 0)),
            pl.BlockSpec((NUM_RQ_STEPS, NUM_TOKENS, TOKEN_DIM),
                         lambda i: (0, 0, 0)),
        ],
        out_specs=[
            pl.BlockSpec((1, 1, BLOCK), lambda i: (i, 0, 0)),
            pl.BlockSpec((1, 1, BLOCK), lambda i: (i, 0, 0)),
            pl.BlockSpec((BLOCK, TOKEN_DIM), lambda i: (i, 0)),
        ],
        out_shape=[
            jax.ShapeDtypeStruct((NUM_BLOCKS, 1, BLOCK), jnp.int32),
            jax.ShapeDtypeStruct((NUM_BLOCKS, 1, BLOCK), jnp.int32),
            jax.ShapeDtypeStruct((TOK_TOTAL, TOKEN_DIM), jnp.float32),
        ],
    )(z_flat, codebooks)
    indices = jnp.stack(
        [idx0.reshape(BATCH, NUM_TOK_PER_IMG),
         idx1.reshape(BATCH, NUM_TOK_PER_IMG)], axis=-1)
    v_q = vq.reshape(BATCH, NUM_TOK_PER_IMG, TOKEN_DIM)
    return (indices, v_q)


# speedup vs baseline: 2.7574x; 2.7574x over previous
import jax
import jax.numpy as jnp
from jax import lax
from jax.experimental import pallas as pl

BATCH = 64
NUM_TOK_PER_IMG = 1024
TOKEN_DIM = 32
NUM_TOKENS = 512
NUM_RQ_STEPS = 2

TOK_TOTAL = BATCH * NUM_TOK_PER_IMG
BLOCK = 4096
NUM_BLOCKS = TOK_TOTAL // BLOCK

_DIMS_KM = (((1,), (0,)), ((), ()))
_DIMS_TT = (((0,), (0,)), ((), ()))


def _vq_step(zt, cb):
    r_sq = jnp.sum(zt * zt, axis=0, keepdims=True)
    c_sq = jnp.sum(cb * cb, axis=-1)[:, None]
    cross = lax.dot_general(cb.astype(jnp.bfloat16),
                            zt.astype(jnp.bfloat16), _DIMS_KM,
                            preferred_element_type=jnp.float32)
    dists = r_sq - 2.0 * cross + c_sq
    m = jnp.min(dists, axis=0, keepdims=True)
    onehot = (dists == m).astype(jnp.bfloat16)
    hi = cb.astype(jnp.bfloat16)
    rem = cb - hi.astype(jnp.float32)
    mid = rem.astype(jnp.bfloat16)
    lo = (rem - mid.astype(jnp.float32)).astype(jnp.bfloat16)
    k = lax.broadcasted_iota(jnp.int32, (NUM_TOKENS, 1), 0)
    k_even = (k - (k % 2)).astype(jnp.bfloat16)
    k_par = (k % 2).astype(jnp.bfloat16)
    parts = jnp.concatenate([hi, mid, lo, k_even, k_par], axis=1)
    q3 = lax.dot_general(parts, onehot, _DIMS_TT,
                         preferred_element_type=jnp.float32)
    q = ((q3[:TOKEN_DIM] + q3[TOKEN_DIM:2 * TOKEN_DIM])
         + q3[2 * TOKEN_DIM:3 * TOKEN_DIM])
    idx = (q3[3 * TOKEN_DIM] + q3[3 * TOKEN_DIM + 1]).astype(jnp.int32)
    return q, idx


def _rvq_block(zt_ref, cb_ref, idx0_ref, idx1_ref, vq_ref):
    zt = zt_ref[...]
    q0, i0 = _vq_step(zt, cb_ref[0])
    q1, i1 = _vq_step(zt - q0, cb_ref[1])
    z_q = q0 + q1
    idx0_ref[0, 0, :] = i0
    idx1_ref[0, 0, :] = i1
    vq_ref[...] = zt + (z_q - zt)


@jax.jit
def kernel(z_e, codebooks):
    zt = z_e.reshape(TOK_TOTAL, TOKEN_DIM).T
    idx0, idx1, vqt = pl.pallas_call(
        _rvq_block,
        grid=(NUM_BLOCKS,),
        in_specs=[
            pl.BlockSpec((TOKEN_DIM, BLOCK), lambda i: (0, i)),
            pl.BlockSpec((NUM_RQ_STEPS, NUM_TOKENS, TOKEN_DIM),
                         lambda i: (0, 0, 0)),
        ],
        out_specs=[
            pl.BlockSpec((1, 1, BLOCK), lambda i: (i, 0, 0)),
            pl.BlockSpec((1, 1, BLOCK), lambda i: (i, 0, 0)),
            pl.BlockSpec((TOKEN_DIM, BLOCK), lambda i: (0, i)),
        ],
        out_shape=[
            jax.ShapeDtypeStruct((NUM_BLOCKS, 1, BLOCK), jnp.int32),
            jax.ShapeDtypeStruct((NUM_BLOCKS, 1, BLOCK), jnp.int32),
            jax.ShapeDtypeStruct((TOKEN_DIM, TOK_TOTAL), jnp.float32),
        ],
    )(zt, codebooks)
    indices = jnp.stack(
        [idx0.reshape(BATCH, NUM_TOK_PER_IMG),
         idx1.reshape(BATCH, NUM_TOK_PER_IMG)], axis=-1)
    v_q = vqt.T.reshape(BATCH, NUM_TOK_PER_IMG, TOKEN_DIM)
    return (indices, v_q)


# speedup vs baseline: 2.9077x; 1.0545x over previous
import jax
import jax.numpy as jnp
from jax import lax
from jax.experimental import pallas as pl

BATCH = 64
NUM_TOK_PER_IMG = 1024
TOKEN_DIM = 32
NUM_TOKENS = 512
NUM_RQ_STEPS = 2

TOK_TOTAL = BATCH * NUM_TOK_PER_IMG
BLOCK = 8192
NUM_BLOCKS = TOK_TOTAL // BLOCK

_DIMS_KM = (((1,), (0,)), ((), ()))
_DIMS_TT = (((0,), (0,)), ((), ()))


def _vq_step(zt, cb):
    r_sq = jnp.sum(zt * zt, axis=0, keepdims=True)
    c_sq = jnp.sum(cb * cb, axis=-1)[:, None]
    cross = lax.dot_general(cb.astype(jnp.bfloat16),
                            zt.astype(jnp.bfloat16), _DIMS_KM,
                            preferred_element_type=jnp.float32)
    dists = r_sq - 2.0 * cross + c_sq
    m = jnp.min(dists, axis=0, keepdims=True)
    onehot = (dists == m).astype(jnp.bfloat16)
    hi = cb.astype(jnp.bfloat16)
    rem = cb - hi.astype(jnp.float32)
    mid = rem.astype(jnp.bfloat16)
    lo = (rem - mid.astype(jnp.float32)).astype(jnp.bfloat16)
    k = lax.broadcasted_iota(jnp.int32, (NUM_TOKENS, 1), 0)
    k_even = (k - (k % 2)).astype(jnp.bfloat16)
    k_par = (k % 2).astype(jnp.bfloat16)
    parts = jnp.concatenate([hi, mid, lo, k_even, k_par], axis=1)
    q3 = lax.dot_general(parts, onehot, _DIMS_TT,
                         preferred_element_type=jnp.float32)
    q = ((q3[:TOKEN_DIM] + q3[TOKEN_DIM:2 * TOKEN_DIM])
         + q3[2 * TOKEN_DIM:3 * TOKEN_DIM])
    idx = (q3[3 * TOKEN_DIM] + q3[3 * TOKEN_DIM + 1]).astype(jnp.int32)
    return q, idx


def _rvq_block(zt_ref, cb_ref, idx0_ref, idx1_ref, vq_ref):
    zt = zt_ref[...]
    q0, i0 = _vq_step(zt, cb_ref[0])
    q1, i1 = _vq_step(zt - q0, cb_ref[1])
    z_q = q0 + q1
    idx0_ref[0, 0, :] = i0
    idx1_ref[0, 0, :] = i1
    vq_ref[...] = zt + (z_q - zt)


@jax.jit
def kernel(z_e, codebooks):
    zt = z_e.reshape(TOK_TOTAL, TOKEN_DIM).T
    idx0, idx1, vqt = pl.pallas_call(
        _rvq_block,
        grid=(NUM_BLOCKS,),
        in_specs=[
            pl.BlockSpec((TOKEN_DIM, BLOCK), lambda i: (0, i)),
            pl.BlockSpec((NUM_RQ_STEPS, NUM_TOKENS, TOKEN_DIM),
                         lambda i: (0, 0, 0)),
        ],
        out_specs=[
            pl.BlockSpec((1, 1, BLOCK), lambda i: (i, 0, 0)),
            pl.BlockSpec((1, 1, BLOCK), lambda i: (i, 0, 0)),
            pl.BlockSpec((TOKEN_DIM, BLOCK), lambda i: (0, i)),
        ],
        out_shape=[
            jax.ShapeDtypeStruct((NUM_BLOCKS, 1, BLOCK), jnp.int32),
            jax.ShapeDtypeStruct((NUM_BLOCKS, 1, BLOCK), jnp.int32),
            jax.ShapeDtypeStruct((TOKEN_DIM, TOK_TOTAL), jnp.float32),
        ],
    )(zt, codebooks)
    indices = jnp.stack(
        [idx0.reshape(BATCH, NUM_TOK_PER_IMG),
         idx1.reshape(BATCH, NUM_TOK_PER_IMG)], axis=-1)
    v_q = vqt.T.reshape(BATCH, NUM_TOK_PER_IMG, TOKEN_DIM)
    return (indices, v_q)
